# Initial kernel scaffold; baseline (speedup 1.0000x reference)
import jax
import jax.numpy as jnp
from jax.experimental import pallas as pl


def kernel(entity_emb, user_emb, relation_emb, edge_index, edge_type, interact_rows, interact_cols, interact_vals, relation_intent_emb):
    raise NotImplementedError("write your pallas kernel here")


# speedup vs baseline: 3.4439x; 3.4439x over previous
import functools

import jax
import jax.numpy as jnp
from jax import lax
from jax.experimental import pallas as pl
from jax.experimental.pallas import tpu as pltpu
from jax.experimental.pallas import tpu_sc as plsc

_NC = 2
_NS = 16
_NW = _NC * _NS
_L = 16


def _make_sc_agg(n_ent, d, e_total, nnz, n_usr, n_rel):
    assert d == 8 * _L
    per_a = e_total // _NW
    ca = 80
    na = per_a // ca
    assert na * ca == per_a
    per_b = nnz // _NW
    cb = 64
    nb = per_b // cb
    assert nb * cb == per_b
    n_ent_p = -(-n_ent // 2048) * 2048
    rze = n_ent_p // _NS
    rzu = n_usr // _NS
    assert rzu * _NS == n_usr and rzu % 8 == 0

    mesh = plsc.VectorSubcoreMesh(core_axis_name="c", subcore_axis_name="s")

    @functools.partial(
        pl.kernel,
        mesh=mesh,
        out_type=(
            jax.ShapeDtypeStruct((_NC, n_ent_p, d), jnp.float32),
            jax.ShapeDtypeStruct((_NC * n_ent_p,), jnp.float32),
            jax.ShapeDtypeStruct((_NC, n_usr, d), jnp.float32),
        ),
        scratch_types=[
            pltpu.VMEM_SHARED((n_ent_p, d), jnp.float32),
            pltpu.VMEM_SHARED((n_ent_p,), jnp.float32),
            pltpu.VMEM((n_rel, d), jnp.float32),
            pltpu.VMEM((ca,), jnp.int32),
            pltpu.VMEM((ca,), jnp.int32),
            pltpu.VMEM((ca,), jnp.int32),
            pltpu.VMEM((ca, d), jnp.float32),
            pltpu.VMEM((ca,), jnp.float32),
            pltpu.VMEM((cb,), jnp.int32),
            pltpu.VMEM((cb,), jnp.int32),
            pltpu.VMEM((cb,), jnp.float32),
            pltpu.VMEM((cb, d), jnp.float32),
            pltpu.SemaphoreType.DMA,
        ],
    )
    def sc_agg(ent_hbm, rel_hbm, tail_hbm, head_hbm, et_hbm,
               urow_hbm, ucol_hbm, uval_hbm, z_main, z_cnt,
               ent_part, cnt_part, usr_part,
               acc, cnt, rel_v, tail_a, head_a, et_a, rows_a,
               ones_v, row_b, col_b, val_b, rows_b, sem):
        c = lax.axis_index("c")
        s = lax.axis_index("s")
        wid = s * _NC + c

        pltpu.sync_copy(rel_hbm, rel_v)
        pltpu.sync_copy(z_main.at[pl.ds(s * rze, rze)],
                        acc.at[pl.ds(s * rze, rze)])
        pltpu.sync_copy(z_cnt.at[pl.ds(s * rze, rze)],
                        cnt.at[pl.ds(s * rze, rze)])

        def fill_ones(i, carry):
            ones_v[pl.ds(i * _L, _L)] = jnp.ones((_L,), jnp.float32)
            return carry
        lax.fori_loop(0, ca // _L, fill_ones, 0)
        plsc.subcore_barrier()

        base_a = wid * per_a

        def chunk_a(i, carry):
            b = base_a + i * ca
            pltpu.sync_copy(tail_hbm.at[pl.ds(b, ca)], tail_a)
            pltpu.sync_copy(head_hbm.at[pl.ds(b, ca)], head_a)
            pltpu.sync_copy(et_hbm.at[pl.ds(b, ca)], et_a)
            pltpu.async_copy(ent_hbm.at[tail_a], rows_a, sem).wait()

            def edge_grp(g, cy):
                et16 = et_a[pl.ds(g * _L, _L)]
                for k in range(_L):
                    e = g * _L + k
                    et = et16[k]
                    for j in range(d // _L):
                        rows_a[e, pl.ds(j * _L, _L)] = (
                            rows_a[e, pl.ds(j * _L, _L)]
                            * rel_v[et, pl.ds(j * _L, _L)])
                return cy
            lax.fori_loop(0, ca // _L, edge_grp, 0)
            pltpu.sync_copy(rows_a, acc.at[head_a], add=True)
            pltpu.sync_copy(ones_v, cnt.at[head_a], add=True)
            return carry
        lax.fori_loop(0, na, chunk_a, 0)
        plsc.subcore_barrier()

        pltpu.sync_copy(acc.at[pl.ds(s * rze, rze)],
                        ent_part.at[c, pl.ds(s * rze, rze)])
        pltpu.sync_copy(cnt.at[pl.ds(s * rze, rze)],
                        cnt_part.at[pl.ds(c * n_ent_p + s * rze, rze)])
        plsc.subcore_barrier()

        pltpu.sync_copy(z_main.at[pl.ds(s * rzu, rzu)],
                        acc.at[pl.ds(s * rzu, rzu)])
        plsc.subcore_barrier()

        base_b = wid * per_b

        def chunk_b(i, carry):
            b = base_b + i * cb
            pltpu.sync_copy(urow_hbm.at[pl.ds(b, cb)], row_b)
            pltpu.sync_copy(ucol_hbm.at[pl.ds(b, cb)], col_b)
            pltpu.sync_copy(uval_hbm.at[pl.ds(b, cb)], val_b)
            pltpu.async_copy(ent_hbm.at[col_b], rows_b, sem).wait()

            def nz_grp(g, cy):
                v16 = val_b[pl.ds(g * _L, _L)]
                for k in range(_L):
                    e = g * _L + k
                    v = v16[k]
                    for j in range(d // _L):
                        rows_b[e, pl.ds(j * _L, _L)] = (
                            rows_b[e, pl.ds(j * _L, _L)] * v)
                return cy
            lax.fori_loop(0, cb // _L, nz_grp, 0)
            pltpu.sync_copy(rows_b, acc.at[row_b], add=True)
            return carry
        lax.fori_loop(0, nb, chunk_b, 0)
        plsc.subcore_barrier()

        pltpu.sync_copy(acc.at[pl.ds(s * rzu, rzu)],
                        usr_part.at[c, pl.ds(s * rzu, rzu)])

    return sc_agg


def _tc_entity(ent_part, cnt_part):
    n_ent, d = ent_part.shape[1], ent_part.shape[2]
    blk = 1024

    def body(ep, cp, o):
        ssum = ep[0] + ep[1]
        cnts = cp[0] + cp[1]
        o[...] = ssum / jnp.maximum(cnts, 1.0)[:, None]

    return pl.pallas_call(
        body,
        grid=(n_ent // blk,),
        in_specs=[
            pl.BlockSpec((2, blk, d), lambda i: (0, i, 0)),
            pl.BlockSpec((2, blk), lambda i: (0, i)),
        ],
        out_specs=pl.BlockSpec((blk, d), lambda i: (i, 0)),
        out_shape=jax.ShapeDtypeStruct((n_ent, d), jnp.float32),
    )(ent_part, cnt_part)


def _tc_user(user_emb, w, usr_part):
    n_usr, d = user_emb.shape
    n_int = w.shape[0]
    blk = 1024

    def body(u, w_, up, o):
        ua = up[0] + up[1]
        score = lax.dot_general(u[...], w_[...], (((1,), (1,)), ((), ())),
                                preferred_element_type=jnp.float32)
        score = score - jnp.max(score, axis=1, keepdims=True)
        score = jnp.exp(score)
        score = score / jnp.sum(score, axis=1, keepdims=True)
        intent = lax.dot_general(score, w_[...], (((1,), (0,)), ((), ())),
                                 preferred_element_type=jnp.float32)
        o[...] = intent * ua + ua

    return pl.pallas_call(
        body,
        grid=(n_usr // blk,),
        in_specs=[
            pl.BlockSpec((blk, d), lambda i: (i, 0)),
            pl.BlockSpec((n_int, d), lambda i: (0, 0)),
            pl.BlockSpec((2, blk, d), lambda i: (0, i, 0)),
        ],
        out_specs=pl.BlockSpec((blk, d), lambda i: (i, 0)),
        out_shape=jax.ShapeDtypeStruct((n_usr, d), jnp.float32),
    )(user_emb, w, usr_part)


def kernel(entity_emb, user_emb, relation_emb, edge_index, edge_type,
           interact_rows, interact_cols, interact_vals, relation_intent_emb):
    n_ent, d = entity_emb.shape
    n_usr = user_emb.shape[0]
    e_total = edge_type.shape[0]
    nnz = interact_rows.shape[0]

    head = edge_index[0].astype(jnp.int32)
    tail = edge_index[1].astype(jnp.int32)
    et = edge_type.astype(jnp.int32)
    urow = interact_rows.astype(jnp.int32)
    ucol = interact_cols.astype(jnp.int32)
    uval = interact_vals.astype(jnp.float32)
    n_ent_p = -(-n_ent // 2048) * 2048
    z_main = jnp.zeros((n_ent_p, d), jnp.float32)
    z_cnt = jnp.zeros((n_ent_p,), jnp.float32)

    sc_agg = _make_sc_agg(n_ent, d, e_total, nnz, n_usr, relation_emb.shape[0])
    ent_part, cnt_part, usr_part = sc_agg(
        entity_emb, relation_emb, tail, head, et, urow, ucol, uval,
        z_main, z_cnt)

    entity_agg = _tc_entity(ent_part, cnt_part.reshape(_NC, n_ent_p))[:n_ent]
    relation_user_agg = _tc_user(user_emb, relation_intent_emb, usr_part)
    return entity_agg, relation_user_agg


# speedup vs baseline: 6.3248x; 1.8365x over previous
import functools

import jax
import jax.numpy as jnp
from jax import lax
from jax.experimental import pallas as pl
from jax.experimental.pallas import tpu as pltpu
from jax.experimental.pallas import tpu_sc as plsc

_NC = 2
_NS = 16
_NW = _NC * _NS
_L = 16

_CA = 80
_CB = 64
_R = 3


def _make_sc_agg(n_ent, d, e_total, nnz, n_usr, n_rel):
    assert d == 8 * _L
    per_a = e_total // _NW
    na = per_a // _CA
    assert na * _CA == per_a
    per_b = nnz // _NW
    nb = per_b // _CB
    assert nb * _CB == per_b
    n_ent_p = -(-n_ent // 2048) * 2048
    rze = n_ent_p // _NS
    rzu = n_usr // _NS
    assert rzu * _NS == n_usr and rzu % 8 == 0
    assert na % _R == _R - 1 and nb % _R == _R - 1

    mesh = plsc.VectorSubcoreMesh(core_axis_name="c", subcore_axis_name="s")

    @functools.partial(
        pl.kernel,
        mesh=mesh,
        out_type=(
            jax.ShapeDtypeStruct((_NC, n_ent_p, d), jnp.float32),
            jax.ShapeDtypeStruct((_NC * n_ent_p,), jnp.float32),
            jax.ShapeDtypeStruct((_NC, n_usr, d), jnp.float32),
        ),
        scratch_types=[
            pltpu.VMEM_SHARED((n_ent_p, d), jnp.float32),
            pltpu.VMEM_SHARED((n_ent_p,), jnp.float32),
            pltpu.VMEM((n_rel, d), jnp.float32),
            pltpu.VMEM((_CA,), jnp.int32),
            pltpu.VMEM((_CB,), jnp.int32),
            pltpu.VMEM((_CA,), jnp.float32),
        ]
        + [pltpu.VMEM((_CA,), jnp.int32) for _ in range(_R)]
        + [pltpu.VMEM((_CA,), jnp.int32) for _ in range(_R)]
        + [pltpu.VMEM((_CA,), jnp.int32) for _ in range(_R)]
        + [pltpu.VMEM((_CB,), jnp.float32) for _ in range(_R)]
        + [pltpu.VMEM((_CA, d), jnp.float32) for _ in range(_R)]
        + [pltpu.SemaphoreType.DMA for _ in range(2 * _R)],
    )
    def sc_agg(ent_hbm, rel_hbm, tail_hbm, head_hbm, et_hbm,
               urow_hbm, ucol_hbm, uval_hbm, z_main, z_cnt,
               ent_part, cnt_part, usr_part,
               acc, cnt, rel_v, head_c, row_c, ones_v, *bufs):
        gidx = list(bufs[:_R])
        sidx = list(bufs[_R:2 * _R])
        etyp = list(bufs[2 * _R:3 * _R])
        vals = list(bufs[3 * _R:4 * _R])
        rows = list(bufs[4 * _R:5 * _R])
        isems = list(bufs[5 * _R:6 * _R])
        gsems = list(bufs[6 * _R:7 * _R])
        c = lax.axis_index("c")
        s = lax.axis_index("s")
        wid = s * _NC + c

        pltpu.sync_copy(rel_hbm, rel_v)
        pltpu.sync_copy(z_main.at[pl.ds(s * rze, rze)],
                        acc.at[pl.ds(s * rze, rze)])
        pltpu.sync_copy(z_cnt.at[pl.ds(s * rze, rze)],
                        cnt.at[pl.ds(s * rze, rze)])

        def fill_ones(i, carry):
            ones_v[pl.ds(i * _L, _L)] = jnp.ones((_L,), jnp.float32)
            return carry
        lax.fori_loop(0, _CA // _L, fill_ones, 0)
        plsc.subcore_barrier()

        base_a = wid * per_a

        def idx_start_a(ci, slot):
            b = base_a + ci * _CA
            pltpu.async_copy(tail_hbm.at[pl.ds(b, _CA)], gidx[slot],
                             isems[slot])
            pltpu.async_copy(head_hbm.at[pl.ds(b, _CA)], sidx[slot],
                             isems[slot])
            pltpu.async_copy(et_hbm.at[pl.ds(b, _CA)], etyp[slot],
                             isems[slot])

        def idx_wait_a(ci, slot):
            b = base_a + ci * _CA
            pltpu.make_async_copy(tail_hbm.at[pl.ds(b, _CA)], gidx[slot],
                                  isems[slot]).wait()
            pltpu.make_async_copy(head_hbm.at[pl.ds(b, _CA)], sidx[slot],
                                  isems[slot]).wait()
            pltpu.make_async_copy(et_hbm.at[pl.ds(b, _CA)], etyp[slot],
                                  isems[slot]).wait()

        def gather_start_a(slot):
            pltpu.async_copy(ent_hbm.at[gidx[slot]], rows[slot], gsems[slot])

        def work_a(slot):
            pltpu.make_async_copy(ent_hbm.at[gidx[slot]], rows[slot],
                                  gsems[slot]).wait()

            def edge_grp(g, cy):
                head_c[pl.ds(g * _L, _L)] = sidx[slot][pl.ds(g * _L, _L)]
                et16 = etyp[slot][pl.ds(g * _L, _L)]
                for k in range(_L):
                    e = g * _L + k
                    et = et16[k]
                    for j in range(d // _L):
                        rows[slot][e, pl.ds(j * _L, _L)] = (
                            rows[slot][e, pl.ds(j * _L, _L)]
                            * rel_v[et, pl.ds(j * _L, _L)])
                return cy
            lax.fori_loop(0, _CA // _L, edge_grp, 0)
            pltpu.sync_copy(rows[slot], acc.at[head_c], add=True)
            pltpu.sync_copy(ones_v, cnt.at[head_c], add=True)

        idx_start_a(0, 0)
        idx_start_a(1, 1)
        idx_wait_a(0, 0)
        gather_start_a(0)

        def loop_a(g, carry):
            for b in range(_R):
                ci = g * _R + b
                idx_start_a(ci + 2, (b + 2) % _R)
                idx_wait_a(ci + 1, (b + 1) % _R)
                gather_start_a((b + 1) % _R)
                work_a(b)
            return carry
        lax.fori_loop(0, na // _R, loop_a, 0)
        idx_wait_a(na - 1, 1)
        gather_start_a(1)
        work_a(0)
        work_a(1)
        plsc.subcore_barrier()

        pltpu.sync_copy(acc.at[pl.ds(s * rze, rze)],
                        ent_part.at[c, pl.ds(s * rze, rze)])
        pltpu.sync_copy(cnt.at[pl.ds(s * rze, rze)],
                        cnt_part.at[pl.ds(c * n_ent_p + s * rze, rze)])
        plsc.subcore_barrier()

        pltpu.sync_copy(z_main.at[pl.ds(s * rzu, rzu)],
                        acc.at[pl.ds(s * rzu, rzu)])
        plsc.subcore_barrier()

        base_b = wid * per_b

        def idx_start_b(ci, slot):
            b = base_b + ci * _CB
            pltpu.async_copy(ucol_hbm.at[pl.ds(b, _CB)],
                             gidx[slot].at[pl.ds(0, _CB)], isems[slot])
            pltpu.async_copy(urow_hbm.at[pl.ds(b, _CB)],
                             sidx[slot].at[pl.ds(0, _CB)], isems[slot])
            pltpu.async_copy(uval_hbm.at[pl.ds(b, _CB)], vals[slot],
                             isems[slot])

        def idx_wait_b(ci, slot):
            b = base_b + ci * _CB
            pltpu.make_async_copy(ucol_hbm.at[pl.ds(b, _CB)],
                                  gidx[slot].at[pl.ds(0, _CB)],
                                  isems[slot]).wait()
            pltpu.make_async_copy(urow_hbm.at[pl.ds(b, _CB)],
                                  sidx[slot].at[pl.ds(0, _CB)],
                                  isems[slot]).wait()
            pltpu.make_async_copy(uval_hbm.at[pl.ds(b, _CB)], vals[slot],
                                  isems[slot]).wait()

        def gather_start_b(slot):
            pltpu.async_copy(ent_hbm.at[gidx[slot].at[pl.ds(0, _CB)]],
                             rows[slot].at[pl.ds(0, _CB)], gsems[slot])

        def work_b(slot):
            pltpu.make_async_copy(ent_hbm.at[gidx[slot].at[pl.ds(0, _CB)]],
                                  rows[slot].at[pl.ds(0, _CB)],
                                  gsems[slot]).wait()

            def nz_grp(g, cy):
                row_c[pl.ds(g * _L, _L)] = sidx[slot][pl.ds(g * _L, _L)]
                v16 = vals[slot][pl.ds(g * _L, _L)]
                for k in range(_L):
                    e = g * _L + k
                    v = v16[k]
                    for j in range(d // _L):
                        rows[slot][e, pl.ds(j * _L, _L)] = (
                            rows[slot][e, pl.ds(j * _L, _L)] * v)
                return cy
            lax.fori_loop(0, _CB // _L, nz_grp, 0)
            pltpu.sync_copy(rows[slot].at[pl.ds(0, _CB)], acc.at[row_c],
                            add=True)

        idx_start_b(0, 0)
        idx_start_b(1, 1)
        idx_wait_b(0, 0)
        gather_start_b(0)

        def loop_b(g, carry):
            for b in range(_R):
                ci = g * _R + b
                idx_start_b(ci + 2, (b + 2) % _R)
                idx_wait_b(ci + 1, (b + 1) % _R)
                gather_start_b((b + 1) % _R)
                work_b(b)
            return carry
        lax.fori_loop(0, nb // _R, loop_b, 0)
        idx_wait_b(nb - 1, 1)
        gather_start_b(1)
        work_b(0)
        work_b(1)
        plsc.subcore_barrier()

        pltpu.sync_copy(acc.at[pl.ds(s * rzu, rzu)],
                        usr_part.at[c, pl.ds(s * rzu, rzu)])

    return sc_agg


def _tc_entity(ent_part, cnt_part):
    n_ent, d = ent_part.shape[1], ent_part.shape[2]
    blk = 1024

    def body(ep, cp, o):
        ssum = ep[0] + ep[1]
        cnts = cp[0] + cp[1]
        o[...] = ssum / jnp.maximum(cnts, 1.0)[:, None]

    return pl.pallas_call(
        body,
        grid=(n_ent // blk,),
        in_specs=[
            pl.BlockSpec((2, blk, d), lambda i: (0, i, 0)),
            pl.BlockSpec((2, blk), lambda i: (0, i)),
        ],
        out_specs=pl.BlockSpec((blk, d), lambda i: (i, 0)),
        out_shape=jax.ShapeDtypeStruct((n_ent, d), jnp.float32),
    )(ent_part, cnt_part)


def _tc_user(user_emb, w, usr_part):
    n_usr, d = user_emb.shape
    n_int = w.shape[0]
    blk = 1024

    def body(u, w_, up, o):
        ua = up[0] + up[1]
        score = lax.dot_general(u[...], w_[...], (((1,), (1,)), ((), ())),
                                preferred_element_type=jnp.float32)
        score = score - jnp.max(score, axis=1, keepdims=True)
        score = jnp.exp(score)
        score = score / jnp.sum(score, axis=1, keepdims=True)
        intent = lax.dot_general(score, w_[...], (((1,), (0,)), ((), ())),
                                 preferred_element_type=jnp.float32)
        o[...] = intent * ua + ua

    return pl.pallas_call(
        body,
        grid=(n_usr // blk,),
        in_specs=[
            pl.BlockSpec((blk, d), lambda i: (i, 0)),
            pl.BlockSpec((n_int, d), lambda i: (0, 0)),
            pl.BlockSpec((2, blk, d), lambda i: (0, i, 0)),
        ],
        out_specs=pl.BlockSpec((blk, d), lambda i: (i, 0)),
        out_shape=jax.ShapeDtypeStruct((n_usr, d), jnp.float32),
    )(user_emb, w, usr_part)


def kernel(entity_emb, user_emb, relation_emb, edge_index, edge_type,
           interact_rows, interact_cols, interact_vals, relation_intent_emb):
    n_ent, d = entity_emb.shape
    n_usr = user_emb.shape[0]
    e_total = edge_type.shape[0]
    nnz = interact_rows.shape[0]

    head = edge_index[0].astype(jnp.int32)
    tail = edge_index[1].astype(jnp.int32)
    et = edge_type.astype(jnp.int32)
    urow = interact_rows.astype(jnp.int32)
    ucol = interact_cols.astype(jnp.int32)
    uval = interact_vals.astype(jnp.float32)
    n_ent_p = -(-n_ent // 2048) * 2048
    z_main = jnp.zeros((n_ent_p, d), jnp.float32)
    z_cnt = jnp.zeros((n_ent_p,), jnp.float32)

    sc_agg = _make_sc_agg(n_ent, d, e_total, nnz, n_usr, relation_emb.shape[0])
    ent_part, cnt_part, usr_part = sc_agg(
        entity_emb, relation_emb, tail, head, et, urow, ucol, uval,
        z_main, z_cnt)

    entity_agg = _tc_entity(ent_part, cnt_part.reshape(_NC, n_ent_p))
    relation_user_agg = _tc_user(user_emb, relation_intent_emb, usr_part)
    return entity_agg[:n_ent], relation_user_agg


# speedup vs baseline: 7.2321x; 1.1435x over previous
import functools

import jax
import jax.numpy as jnp
from jax import lax
from jax.experimental import pallas as pl
from jax.experimental.pallas import tpu as pltpu
from jax.experimental.pallas import tpu_sc as plsc

_NC = 2
_NS = 16
_NW = _NC * _NS
_L = 16

_CA = 80
_CB = 64
_R = 3


def _make_sc_agg(n_ent, d, e_total, nnz, n_usr, n_rel):
    assert d == 8 * _L
    per_a = e_total // _NW
    na = per_a // _CA
    assert na * _CA == per_a
    per_b = nnz // _NW
    nb = per_b // _CB
    assert nb * _CB == per_b
    n_ent_p = -(-n_ent // 2048) * 2048
    rze = n_ent_p // _NS
    rzu = n_usr // _NS
    assert rzu * _NS == n_usr and rzu % 8 == 0
    assert na % _R == _R - 1 and nb % _R == _R - 1

    mesh = plsc.VectorSubcoreMesh(core_axis_name="c", subcore_axis_name="s")

    @functools.partial(
        pl.kernel,
        mesh=mesh,
        out_type=(
            jax.ShapeDtypeStruct((_NC, n_ent_p, d), jnp.float32),
            jax.ShapeDtypeStruct((_NC * n_ent_p,), jnp.float32),
            jax.ShapeDtypeStruct((_NC, n_usr, d), jnp.float32),
        ),
        scratch_types=[
            pltpu.VMEM_SHARED((n_ent_p, d), jnp.float32),
            pltpu.VMEM_SHARED((n_ent_p,), jnp.float32),
            pltpu.VMEM((n_rel, d), jnp.float32),
            pltpu.VMEM((_CA,), jnp.float32),
        ]
        + [pltpu.VMEM((_CA,), jnp.int32) for _ in range(_R)]
        + [pltpu.VMEM((_CA,), jnp.int32) for _ in range(_R)]
        + [pltpu.VMEM((_CA,), jnp.int32) for _ in range(_R)]
        + [pltpu.VMEM((_CB,), jnp.float32) for _ in range(_R)]
        + [pltpu.VMEM((_CA, d), jnp.float32) for _ in range(_R)]
        + [pltpu.VMEM((_CA,), jnp.int32) for _ in range(_R)]
        + [pltpu.VMEM((_CB,), jnp.int32) for _ in range(_R)]
        + [pltpu.SemaphoreType.DMA for _ in range(3 * _R)],
    )
    def sc_agg(ent_hbm, rel_hbm, tail_hbm, head_hbm, et_hbm,
               urow_hbm, ucol_hbm, uval_hbm, z_main, z_cnt,
               ent_part, cnt_part, usr_part,
               acc, cnt, rel_v, ones_v, *bufs):
        gidx = list(bufs[:_R])
        sidx = list(bufs[_R:2 * _R])
        etyp = list(bufs[2 * _R:3 * _R])
        vals = list(bufs[3 * _R:4 * _R])
        rows = list(bufs[4 * _R:5 * _R])
        hidx = list(bufs[5 * _R:6 * _R])
        ridx = list(bufs[6 * _R:7 * _R])
        isems = list(bufs[7 * _R:8 * _R])
        gsems = list(bufs[8 * _R:9 * _R])
        ssems = list(bufs[9 * _R:10 * _R])
        c = lax.axis_index("c")
        s = lax.axis_index("s")
        wid = s * _NC + c

        pltpu.sync_copy(rel_hbm, rel_v)
        pltpu.sync_copy(z_main.at[pl.ds(s * rze, rze)],
                        acc.at[pl.ds(s * rze, rze)])
        pltpu.sync_copy(z_cnt.at[pl.ds(s * rze, rze)],
                        cnt.at[pl.ds(s * rze, rze)])

        def fill_ones(i, carry):
            ones_v[pl.ds(i * _L, _L)] = jnp.ones((_L,), jnp.float32)
            return carry
        lax.fori_loop(0, _CA // _L, fill_ones, 0)
        plsc.subcore_barrier()

        base_a = wid * per_a

        def idx_start_a(ci, slot):
            b = base_a + ci * _CA
            pltpu.async_copy(tail_hbm.at[pl.ds(b, _CA)], gidx[slot],
                             isems[slot])
            pltpu.async_copy(head_hbm.at[pl.ds(b, _CA)], sidx[slot],
                             isems[slot])
            pltpu.async_copy(et_hbm.at[pl.ds(b, _CA)], etyp[slot],
                             isems[slot])

        def idx_wait_a(ci, slot):
            b = base_a + ci * _CA
            pltpu.make_async_copy(tail_hbm.at[pl.ds(b, _CA)], gidx[slot],
                                  isems[slot]).wait()
            pltpu.make_async_copy(head_hbm.at[pl.ds(b, _CA)], sidx[slot],
                                  isems[slot]).wait()
            pltpu.make_async_copy(et_hbm.at[pl.ds(b, _CA)], etyp[slot],
                                  isems[slot]).wait()

        def gather_start_a(slot):
            pltpu.async_copy(ent_hbm.at[gidx[slot]], rows[slot], gsems[slot])

        def scat_start_a(slot):
            pltpu.async_copy(rows[slot], acc.at[hidx[slot]], ssems[slot],
                             add=True)
            pltpu.async_copy(ones_v, cnt.at[hidx[slot]], ssems[slot],
                             add=True)

        def scat_wait_a(slot):
            pltpu.make_async_copy(rows[slot], acc.at[hidx[slot]],
                                  ssems[slot]).wait()
            pltpu.make_async_copy(ones_v, cnt.at[hidx[slot]],
                                  ssems[slot]).wait()

        def work_a(slot):
            pltpu.make_async_copy(ent_hbm.at[gidx[slot]], rows[slot],
                                  gsems[slot]).wait()

            def edge_grp(g, cy):
                hidx[slot][pl.ds(g * _L, _L)] = sidx[slot][pl.ds(g * _L, _L)]
                et16 = etyp[slot][pl.ds(g * _L, _L)]
                for k in range(_L):
                    e = g * _L + k
                    et = et16[k]
                    for j in range(d // _L):
                        rows[slot][e, pl.ds(j * _L, _L)] = (
                            rows[slot][e, pl.ds(j * _L, _L)]
                            * rel_v[et, pl.ds(j * _L, _L)])
                return cy
            lax.fori_loop(0, _CA // _L, edge_grp, 0)
            scat_start_a(slot)

        idx_start_a(0, 0)
        idx_start_a(1, 1)
        idx_wait_a(0, 0)
        gather_start_a(0)

        def loop_a(g, carry):
            for b in range(_R):
                ci = g * _R + b
                idx_start_a(ci + 2, (b + 2) % _R)
                idx_wait_a(ci + 1, (b + 1) % _R)

                @pl.when(ci >= 2)
                def _():
                    scat_wait_a((b + 1) % _R)
                gather_start_a((b + 1) % _R)
                work_a(b)
            return carry
        lax.fori_loop(0, na // _R, loop_a, 0)
        idx_wait_a(na - 1, 1)
        scat_wait_a(1)
        gather_start_a(1)
        work_a(0)
        work_a(1)
        scat_wait_a(2)
        scat_wait_a(0)
        scat_wait_a(1)
        plsc.subcore_barrier()

        pltpu.sync_copy(acc.at[pl.ds(s * rze, rze)],
                        ent_part.at[c, pl.ds(s * rze, rze)])
        pltpu.sync_copy(cnt.at[pl.ds(s * rze, rze)],
                        cnt_part.at[pl.ds(c * n_ent_p + s * rze, rze)])
        plsc.subcore_barrier()

        pltpu.sync_copy(z_main.at[pl.ds(s * rzu, rzu)],
                        acc.at[pl.ds(s * rzu, rzu)])
        plsc.subcore_barrier()

        base_b = wid * per_b

        def idx_start_b(ci, slot):
            b = base_b + ci * _CB
            pltpu.async_copy(ucol_hbm.at[pl.ds(b, _CB)],
                             gidx[slot].at[pl.ds(0, _CB)], isems[slot])
            pltpu.async_copy(urow_hbm.at[pl.ds(b, _CB)],
                             sidx[slot].at[pl.ds(0, _CB)], isems[slot])
            pltpu.async_copy(uval_hbm.at[pl.ds(b, _CB)], vals[slot],
                             isems[slot])

        def idx_wait_b(ci, slot):
            b = base_b + ci * _CB
            pltpu.make_async_copy(ucol_hbm.at[pl.ds(b, _CB)],
                                  gidx[slot].at[pl.ds(0, _CB)],
                                  isems[slot]).wait()
            pltpu.make_async_copy(urow_hbm.at[pl.ds(b, _CB)],
                                  sidx[slot].at[pl.ds(0, _CB)],
                                  isems[slot]).wait()
            pltpu.make_async_copy(uval_hbm.at[pl.ds(b, _CB)], vals[slot],
                                  isems[slot]).wait()

        def gather_start_b(slot):
            pltpu.async_copy(ent_hbm.at[gidx[slot].at[pl.ds(0, _CB)]],
                             rows[slot].at[pl.ds(0, _CB)], gsems[slot])

        def scat_start_b(slot):
            pltpu.async_copy(rows[slot].at[pl.ds(0, _CB)], acc.at[ridx[slot]],
                             ssems[slot], add=True)

        def scat_wait_b(slot):
            pltpu.make_async_copy(rows[slot].at[pl.ds(0, _CB)],
                                  acc.at[ridx[slot]], ssems[slot]).wait()

        def work_b(slot):
            pltpu.make_async_copy(ent_hbm.at[gidx[slot].at[pl.ds(0, _CB)]],
                                  rows[slot].at[pl.ds(0, _CB)],
                                  gsems[slot]).wait()

            def nz_grp(g, cy):
                ridx[slot][pl.ds(g * _L, _L)] = sidx[slot][pl.ds(g * _L, _L)]
                v16 = vals[slot][pl.ds(g * _L, _L)]
                for k in range(_L):
                    e = g * _L + k
                    v = v16[k]
                    for j in range(d // _L):
                        rows[slot][e, pl.ds(j * _L, _L)] = (
                            rows[slot][e, pl.ds(j * _L, _L)] * v)
                return cy
            lax.fori_loop(0, _CB // _L, nz_grp, 0)
            scat_start_b(slot)

        idx_start_b(0, 0)
        idx_start_b(1, 1)
        idx_wait_b(0, 0)
        gather_start_b(0)

        def loop_b(g, carry):
            for b in range(_R):
                ci = g * _R + b
                idx_start_b(ci + 2, (b + 2) % _R)
                idx_wait_b(ci + 1, (b + 1) % _R)

                @pl.when(ci >= 2)
                def _():
                    scat_wait_b((b + 1) % _R)
                gather_start_b((b + 1) % _R)
                work_b(b)
            return carry
        lax.fori_loop(0, nb // _R, loop_b, 0)
        idx_wait_b(nb - 1, 1)
        scat_wait_b(1)
        gather_start_b(1)
        work_b(0)
        work_b(1)
        scat_wait_b(2)
        scat_wait_b(0)
        scat_wait_b(1)
        plsc.subcore_barrier()

        pltpu.sync_copy(acc.at[pl.ds(s * rzu, rzu)],
                        usr_part.at[c, pl.ds(s * rzu, rzu)])

    return sc_agg


def _tc_entity(ent_part, cnt_part):
    n_ent, d = ent_part.shape[1], ent_part.shape[2]
    blk = 1024

    def body(ep, cp, o):
        ssum = ep[0] + ep[1]
        cnts = cp[0] + cp[1]
        o[...] = ssum / jnp.maximum(cnts, 1.0)[:, None]

    return pl.pallas_call(
        body,
        grid=(n_ent // blk,),
        in_specs=[
            pl.BlockSpec((2, blk, d), lambda i: (0, i, 0)),
            pl.BlockSpec((2, blk), lambda i: (0, i)),
        ],
        out_specs=pl.BlockSpec((blk, d), lambda i: (i, 0)),
        out_shape=jax.ShapeDtypeStruct((n_ent, d), jnp.float32),
    )(ent_part, cnt_part)


def _tc_user(user_emb, w, usr_part):
    n_usr, d = user_emb.shape
    n_int = w.shape[0]
    blk = 1024

    def body(u, w_, up, o):
        ua = up[0] + up[1]
        score = lax.dot_general(u[...], w_[...], (((1,), (1,)), ((), ())),
                                preferred_element_type=jnp.float32)
        score = score - jnp.max(score, axis=1, keepdims=True)
        score = jnp.exp(score)
        score = score / jnp.sum(score, axis=1, keepdims=True)
        intent = lax.dot_general(score, w_[...], (((1,), (0,)), ((), ())),
                                 preferred_element_type=jnp.float32)
        o[...] = intent * ua + ua

    return pl.pallas_call(
        body,
        grid=(n_usr // blk,),
        in_specs=[
            pl.BlockSpec((blk, d), lambda i: (i, 0)),
            pl.BlockSpec((n_int, d), lambda i: (0, 0)),
            pl.BlockSpec((2, blk, d), lambda i: (0, i, 0)),
        ],
        out_specs=pl.BlockSpec((blk, d), lambda i: (i, 0)),
        out_shape=jax.ShapeDtypeStruct((n_usr, d), jnp.float32),
    )(user_emb, w, usr_part)


def kernel(entity_emb, user_emb, relation_emb, edge_index, edge_type,
           interact_rows, interact_cols, interact_vals, relation_intent_emb):
    n_ent, d = entity_emb.shape
    n_usr = user_emb.shape[0]
    e_total = edge_type.shape[0]
    nnz = interact_rows.shape[0]

    head = edge_index[0].astype(jnp.int32)
    tail = edge_index[1].astype(jnp.int32)
    et = edge_type.astype(jnp.int32)
    urow = interact_rows.astype(jnp.int32)
    ucol = interact_cols.astype(jnp.int32)
    uval = interact_vals.astype(jnp.float32)
    n_ent_p = -(-n_ent // 2048) * 2048
    z_main = jnp.zeros((n_ent_p, d), jnp.float32)
    z_cnt = jnp.zeros((n_ent_p,), jnp.float32)

    sc_agg = _make_sc_agg(n_ent, d, e_total, nnz, n_usr, relation_emb.shape[0])
    ent_part, cnt_part, usr_part = sc_agg(
        entity_emb, relation_emb, tail, head, et, urow, ucol, uval,
        z_main, z_cnt)

    entity_agg = _tc_entity(ent_part, cnt_part.reshape(_NC, n_ent_p))
    relation_user_agg = _tc_user(user_emb, relation_intent_emb, usr_part)
    return entity_agg[:n_ent], relation_user_agg
